# baseline (speedup 1.0000x reference)
import functools

import jax
import jax.numpy as jnp
from jax import lax
from jax.experimental import pallas as pl
from jax.experimental.pallas import tpu as pltpu
from jax.experimental.pallas import tpu_sc as plsc

B, T, F, D = 4096, 50, 4, 32
NC, NS = 2, 16
NW = NC * NS
PB = B // NW
NB = 4
NCHUNK = PB // NB
FH_PER = NB * T * F
TK_PER = NB * T
IH = PB // 2


def _chunks(total):
  out, off = [], 0
  while off < total:
    sz = min(128, total - off)
    out.append((off, sz))
    off += sz
  return out

_mesh = plsc.VectorSubcoreMesh(core_axis_name="c", subcore_axis_name="s")


@functools.partial(
    pl.kernel,
    out_type=[
        jax.ShapeDtypeStruct((B, 3 * D), jnp.float32),
        jax.ShapeDtypeStruct((B * T, 2 * D), jnp.float32),
        jax.ShapeDtypeStruct((B, 2 * D), jnp.float32),
    ],
    mesh=_mesh,
    compiler_params=pltpu.CompilerParams(use_tc_tiling_on_sc=False),
    scratch_types=[
        pltpu.VMEM((FH_PER,), jnp.int32),
        pltpu.VMEM((TK_PER + 8,), jnp.int32),
        pltpu.VMEM((TK_PER + 8,), jnp.int32),
        pltpu.VMEM((IH,), jnp.int32),
        pltpu.VMEM((IH,), jnp.int32),
        pltpu.VMEM((IH * F,), jnp.int32),
        pltpu.VMEM((FH_PER, D), jnp.float32),
        pltpu.VMEM((TK_PER, 4 * D), jnp.float32),
        pltpu.VMEM((TK_PER, 2 * D), jnp.float32),
        pltpu.VMEM((PB, 2 * D), jnp.float32),
        pltpu.VMEM((IH, D), jnp.float32),
        pltpu.VMEM((IH, D), jnp.float32),
        pltpu.VMEM((IH * F, D), jnp.float32),
        pltpu.VMEM((IH, 3 * D), jnp.float32),
        pltpu.SemaphoreType.DMA,
    ],
)
def _sc_body(fh_idx, tk_idx, nm_idx, nmh_idx, fb_idx, ftab, ttab, ntab,
             ieb_out, his_out, hsum_out,
             fidx_v, tidx_v, tlin_v, nidx_v, nhidx_v, fbidx_v,
             fg, tg, hisb, hsum_w, n1, n2, fbg, iebb, sem):
  wid = lax.axis_index("s") * NC + lax.axis_index("c")
  wbase = wid * PB

  for j in range(2):
    hb = wbase + j * IH
    pltpu.sync_copy(nm_idx.at[pl.ds(hb, IH)], nidx_v)
    pltpu.sync_copy(nmh_idx.at[pl.ds(hb, IH)], nhidx_v)
    pltpu.sync_copy(fb_idx.at[pl.ds(hb * F, IH * F)], fbidx_v)
    cps = [pltpu.async_copy(ntab.at[nidx_v], n1, sem),
           pltpu.async_copy(ntab.at[nhidx_v], n2, sem)]
    for off, sz in _chunks(IH * F):
      cps.append(pltpu.async_copy(ftab.at[fbidx_v.at[pl.ds(off, sz)]],
                                  fbg.at[pl.ds(off, sz)], sem))
    for cp in cps:
      cp.wait()

    def ibody(b, c):
      for h in (0, 16):
        iebb[b, pl.ds(h, 16)] = n1[b, pl.ds(h, 16)]
        iebb[b, pl.ds(D + h, 16)] = n2[b, pl.ds(h, 16)]
        fv = (fbg[4 * b, pl.ds(h, 16)] + fbg[4 * b + 1, pl.ds(h, 16)] +
              fbg[4 * b + 2, pl.ds(h, 16)] + fbg[4 * b + 3, pl.ds(h, 16)])
        iebb[b, pl.ds(2 * D + h, 16)] = fv
      return c
    lax.fori_loop(0, IH, ibody, 0)
    pltpu.sync_copy(iebb, ieb_out.at[pl.ds(hb, IH)])

  zero = jnp.zeros((16,), jnp.float32)

  def zbody(b, c):
    for h in range(0, 2 * D, 16):
      hsum_w[b, pl.ds(h, 16)] = zero
    return c
  lax.fori_loop(0, PB, zbody, 0)

  def subchunk(si, carry):
    base = wbase + si * NB
    pltpu.sync_copy(fh_idx.at[pl.ds(base * T * F, FH_PER)], fidx_v)
    pltpu.sync_copy(tk_idx.at[pl.ds(base * T, TK_PER)], tidx_v.at[pl.ds(0, TK_PER)])

    def linbody(i, c):
      tv = tidx_v[pl.ds(i * 16, 16)]
      tlin_v[pl.ds(i * 16, 16)] = lax.shift_right_logical(tv, 2)
      return c
    lax.fori_loop(0, (TK_PER + 15) // 16, linbody, 0)

    cps = []
    for off, sz in _chunks(FH_PER):
      cps.append(pltpu.async_copy(ftab.at[fidx_v.at[pl.ds(off, sz)]],
                                  fg.at[pl.ds(off, sz)], sem))
    for off, sz in _chunks(TK_PER):
      cps.append(pltpu.async_copy(ttab.at[tlin_v.at[pl.ds(off, sz)]],
                                  tg.at[pl.ds(off, sz)], sem))
    for cp in cps:
      cp.wait()

    def pbody(g, c):
      p0 = g * 8
      tvv = tidx_v[pl.ds(p0, 16)]
      for j in range(8):
        p = p0 + j
        bb = si * NB + p // T
        col = (tvv[j] & 3) * D
        for h in (0, 16):
          v = (fg[4 * p, pl.ds(h, 16)] + fg[4 * p + 1, pl.ds(h, 16)] +
               fg[4 * p + 2, pl.ds(h, 16)] + fg[4 * p + 3, pl.ds(h, 16)])
          hisb[p, pl.ds(h, 16)] = v
          plsc.addupdate(hsum_w.at[bb, pl.ds(h, 16)], v)
          w = tg[p, pl.ds(col + h, 16)]
          hisb[p, pl.ds(D + h, 16)] = w
          plsc.addupdate(hsum_w.at[bb, pl.ds(D + h, 16)], w)
      return c
    lax.fori_loop(0, TK_PER // 8, pbody, 0)

    pltpu.sync_copy(hisb, his_out.at[pl.ds(base * T, TK_PER)])
    return carry

  lax.fori_loop(0, NCHUNK, subchunk, 0)
  pltpu.sync_copy(hsum_w, hsum_out.at[pl.ds(wbase, PB)])


@jax.jit
def kernel(nameid_batch, funcid_batch, nameid_his_batch, funcid_his_batch,
           trackid_his_batch, nameid_emb, funcid_emb, trackid_emb):
  ieb, his, hsum = _sc_body(
      funcid_his_batch.reshape(B * T * F),
      trackid_his_batch.reshape(B * T),
      nameid_batch,
      nameid_his_batch,
      funcid_batch.reshape(B * F),
      funcid_emb, trackid_emb.reshape(-1, 4 * D), nameid_emb)
  return ieb, his.reshape(B, T, 2 * D), hsum


# speedup vs baseline: 1.2336x; 1.2336x over previous
import functools

import jax
import jax.numpy as jnp
from jax import lax
from jax.experimental import pallas as pl
from jax.experimental.pallas import tpu as pltpu
from jax.experimental.pallas import tpu_sc as plsc

B, T, F, D = 4096, 50, 4, 32
NC, NS = 2, 16
NW = NC * NS
PB = B // NW
NB = 4
NCHUNK = PB // NB
NPAIR = NCHUNK // 2
FH_PER = NB * T * F
TK_PER = NB * T
IH = PB // 2


def _chunks(total):
  out, off = [], 0
  while off < total:
    sz = min(128, total - off)
    out.append((off, sz))
    off += sz
  return out

_mesh = plsc.VectorSubcoreMesh(core_axis_name="c", subcore_axis_name="s")


@functools.partial(
    pl.kernel,
    out_type=[
        jax.ShapeDtypeStruct((B, 3 * D), jnp.float32),
        jax.ShapeDtypeStruct((B * T, 2 * D), jnp.float32),
        jax.ShapeDtypeStruct((B, 2 * D), jnp.float32),
    ],
    mesh=_mesh,
    compiler_params=pltpu.CompilerParams(use_tc_tiling_on_sc=False),
    scratch_types=[
        pltpu.VMEM((FH_PER,), jnp.int32),
        pltpu.VMEM((FH_PER,), jnp.int32),
        pltpu.VMEM((TK_PER,), jnp.int32),
        pltpu.VMEM((TK_PER,), jnp.int32),
        pltpu.VMEM((PB,), jnp.int32),
        pltpu.VMEM((PB,), jnp.int32),
        pltpu.VMEM((PB * F,), jnp.int32),
        pltpu.VMEM((FH_PER, D), jnp.float32),
        pltpu.VMEM((FH_PER, D), jnp.float32),
        pltpu.VMEM((TK_PER, D), jnp.float32),
        pltpu.VMEM((TK_PER, D), jnp.float32),
        pltpu.VMEM((TK_PER, 2 * D), jnp.float32),
        pltpu.VMEM((TK_PER, 2 * D), jnp.float32),
        pltpu.VMEM((2 * NB, 2 * D), jnp.float32),
        pltpu.VMEM((PB, D), jnp.float32),
        pltpu.VMEM((PB, D), jnp.float32),
        pltpu.VMEM((PB * F, D), jnp.float32),
        pltpu.VMEM((IH, 3 * D), jnp.float32),
        pltpu.SemaphoreType.DMA,
        pltpu.SemaphoreType.DMA,
        pltpu.SemaphoreType.DMA,
        pltpu.SemaphoreType.DMA,
        pltpu.SemaphoreType.DMA,
        pltpu.SemaphoreType.DMA,
    ],
)
def _sc_body(fh_idx, tk_idx, nm_idx, nmh_idx, fb_idx, ftab, ttab, ntab,
             ieb_out, his_out, hsum_out,
             fidx0, fidx1, tidx0, tidx1, nidx_v, nhidx_v, fbidx_v,
             fg0, fg1, tg0, tg1, hisb0, hisb1, hsumb,
             n1, n2, fbg, iebb,
             semA, semB, semWA, semWB, semH, semE):
  wid = lax.axis_index("s") * NC + lax.axis_index("c")
  wbase = wid * PB
  fidx = (fidx0, fidx1)
  tidx = (tidx0, tidx1)
  fg = (fg0, fg1)
  tg = (tg0, tg1)
  hisb = (hisb0, hisb1)
  semG = (semA, semB)
  semW = (semWA, semWB)
  zero = jnp.zeros((16,), jnp.float32)

  def fire(si, buf):
    base = wbase + si * NB
    pltpu.sync_copy(fh_idx.at[pl.ds(base * T * F, FH_PER)], fidx[buf])
    pltpu.sync_copy(tk_idx.at[pl.ds(base * T, TK_PER)], tidx[buf])
    for off, sz in _chunks(FH_PER):
      pltpu.async_copy(ftab.at[fidx[buf].at[pl.ds(off, sz)]],
                       fg[buf].at[pl.ds(off, sz)], semG[buf])
    for off, sz in _chunks(TK_PER):
      pltpu.async_copy(ttab.at[tidx[buf].at[pl.ds(off, sz)]],
                       tg[buf].at[pl.ds(off, sz)], semG[buf])

  def drain_gathers(buf):
    pltpu.make_async_copy(ftab.at[pl.ds(0, FH_PER)], fg[buf], semG[buf]).wait()
    pltpu.make_async_copy(ttab.at[pl.ds(0, TK_PER)], tg[buf], semG[buf]).wait()

  def compute_chunk(si, buf, hrow):
    fgb, tgb, hb = fg[buf], tg[buf], hisb[buf]

    for b in range(NB):
      def tbody(t, acc):
        a0, a1, a2, a3 = acc
        p = b * T + t
        v0 = (fgb[4 * p, pl.ds(0, 16)] + fgb[4 * p + 1, pl.ds(0, 16)] +
              fgb[4 * p + 2, pl.ds(0, 16)] + fgb[4 * p + 3, pl.ds(0, 16)])
        v1 = (fgb[4 * p, pl.ds(16, 16)] + fgb[4 * p + 1, pl.ds(16, 16)] +
              fgb[4 * p + 2, pl.ds(16, 16)] + fgb[4 * p + 3, pl.ds(16, 16)])
        w0 = tgb[p, pl.ds(0, 16)]
        w1 = tgb[p, pl.ds(16, 16)]
        hb[p, pl.ds(0, 16)] = v0
        hb[p, pl.ds(16, 16)] = v1
        hb[p, pl.ds(32, 16)] = w0
        hb[p, pl.ds(48, 16)] = w1
        return (a0 + v0, a1 + v1, a2 + w0, a3 + w1)
      a0, a1, a2, a3 = lax.fori_loop(0, T, tbody, (zero, zero, zero, zero))
      hsumb[hrow + b, pl.ds(0, 16)] = a0
      hsumb[hrow + b, pl.ds(16, 16)] = a1
      hsumb[hrow + b, pl.ds(32, 16)] = a2
      hsumb[hrow + b, pl.ds(48, 16)] = a3

    pltpu.async_copy(hb, his_out.at[pl.ds((wbase + si * NB) * T, TK_PER)],
                     semW[buf])

  pltpu.sync_copy(nm_idx.at[pl.ds(wbase, PB)], nidx_v)
  pltpu.sync_copy(nmh_idx.at[pl.ds(wbase, PB)], nhidx_v)
  pltpu.sync_copy(fb_idx.at[pl.ds(wbase * F, PB * F)], fbidx_v)
  pltpu.async_copy(ntab.at[nidx_v], n1, semE)
  pltpu.async_copy(ntab.at[nhidx_v], n2, semE)
  for off, sz in _chunks(PB * F):
    pltpu.async_copy(ftab.at[fbidx_v.at[pl.ds(off, sz)]],
                     fbg.at[pl.ds(off, sz)], semE)
  fire(0, 0)
  fire(1, 1)

  pltpu.make_async_copy(ntab.at[pl.ds(0, PB)], n1, semE).wait()
  pltpu.make_async_copy(ntab.at[pl.ds(0, PB)], n2, semE).wait()
  pltpu.make_async_copy(ftab.at[pl.ds(0, PB * F)], fbg, semE).wait()
  for j in range(2):
    def ibody(r, c):
      b = j * IH + r
      for h in (0, 16):
        iebb[r, pl.ds(h, 16)] = n1[b, pl.ds(h, 16)]
        iebb[r, pl.ds(D + h, 16)] = n2[b, pl.ds(h, 16)]
        fv = (fbg[4 * b, pl.ds(h, 16)] + fbg[4 * b + 1, pl.ds(h, 16)] +
              fbg[4 * b + 2, pl.ds(h, 16)] + fbg[4 * b + 3, pl.ds(h, 16)])
        iebb[r, pl.ds(2 * D + h, 16)] = fv
      return c
    lax.fori_loop(0, IH, ibody, 0)
    pltpu.sync_copy(iebb, ieb_out.at[pl.ds(wbase + j * IH, IH)])

  def pairbody(pi, carry):
    @pl.when(pi > 0)
    def _():
      pltpu.make_async_copy(hisb[0],
                            his_out.at[pl.ds(wbase * T, TK_PER)],
                            semW[0]).wait()
      pltpu.make_async_copy(hsumb,
                            hsum_out.at[pl.ds(wbase, 2 * NB)],
                            semH).wait()
    drain_gathers(0)
    compute_chunk(2 * pi, 0, 0)

    @pl.when(pi < NPAIR - 1)
    def _():
      fire(2 * pi + 2, 0)

    @pl.when(pi > 0)
    def _():
      pltpu.make_async_copy(hisb[1],
                            his_out.at[pl.ds(wbase * T, TK_PER)],
                            semW[1]).wait()
    drain_gathers(1)
    compute_chunk(2 * pi + 1, 1, NB)
    pltpu.async_copy(hsumb, hsum_out.at[pl.ds(wbase + pi * 2 * NB, 2 * NB)],
                     semH)

    @pl.when(pi < NPAIR - 1)
    def _():
      fire(2 * pi + 3, 1)
    return carry

  lax.fori_loop(0, NPAIR, pairbody, 0)

  pltpu.make_async_copy(hisb[0], his_out.at[pl.ds(wbase * T, TK_PER)],
                        semW[0]).wait()
  pltpu.make_async_copy(hisb[1], his_out.at[pl.ds(wbase * T, TK_PER)],
                        semW[1]).wait()
  pltpu.make_async_copy(hsumb, hsum_out.at[pl.ds(wbase, 2 * NB)],
                        semH).wait()


@jax.jit
def kernel(nameid_batch, funcid_batch, nameid_his_batch, funcid_his_batch,
           trackid_his_batch, nameid_emb, funcid_emb, trackid_emb):
  ieb, his, hsum = _sc_body(
      funcid_his_batch.reshape(B * T * F),
      trackid_his_batch.reshape(B * T),
      nameid_batch,
      nameid_his_batch,
      funcid_batch.reshape(B * F),
      funcid_emb, trackid_emb, nameid_emb)
  return ieb, his.reshape(B, T, 2 * D), hsum


# speedup vs baseline: 1.2355x; 1.0015x over previous
import functools

import jax
import jax.numpy as jnp
from jax import lax
from jax.experimental import pallas as pl
from jax.experimental.pallas import tpu as pltpu
from jax.experimental.pallas import tpu_sc as plsc

B, T, F, D = 4096, 50, 4, 32
NC, NS = 2, 16
NW = NC * NS
PB = B // NW
NB = 4
NCHUNK = PB // NB
NPAIR = NCHUNK // 2
FH_PER = NB * T * F
TK_PER = NB * T
IH = PB // 2


def _chunks(total):
  out, off = [], 0
  while off < total:
    sz = min(128, total - off)
    out.append((off, sz))
    off += sz
  return out

_mesh = plsc.VectorSubcoreMesh(core_axis_name="c", subcore_axis_name="s")


@functools.partial(
    pl.kernel,
    out_type=[
        jax.ShapeDtypeStruct((B, 3 * D), jnp.float32),
        jax.ShapeDtypeStruct((B, T, 2 * D), jnp.float32),
        jax.ShapeDtypeStruct((B, 2 * D), jnp.float32),
    ],
    mesh=_mesh,
    compiler_params=pltpu.CompilerParams(use_tc_tiling_on_sc=False),
    scratch_types=[
        pltpu.VMEM((FH_PER,), jnp.int32),
        pltpu.VMEM((FH_PER,), jnp.int32),
        pltpu.VMEM((TK_PER,), jnp.int32),
        pltpu.VMEM((TK_PER,), jnp.int32),
        pltpu.VMEM((PB,), jnp.int32),
        pltpu.VMEM((PB,), jnp.int32),
        pltpu.VMEM((PB * F,), jnp.int32),
        pltpu.VMEM((FH_PER, D), jnp.float32),
        pltpu.VMEM((FH_PER, D), jnp.float32),
        pltpu.VMEM((TK_PER, D), jnp.float32),
        pltpu.VMEM((TK_PER, D), jnp.float32),
        pltpu.VMEM((NB, T, 2 * D), jnp.float32),
        pltpu.VMEM((NB, T, 2 * D), jnp.float32),
        pltpu.VMEM((2 * NB, 2 * D), jnp.float32),
        pltpu.VMEM((PB, D), jnp.float32),
        pltpu.VMEM((PB, D), jnp.float32),
        pltpu.VMEM((PB * F, D), jnp.float32),
        pltpu.VMEM((IH, 3 * D), jnp.float32),
        pltpu.SemaphoreType.DMA,
        pltpu.SemaphoreType.DMA,
        pltpu.SemaphoreType.DMA,
        pltpu.SemaphoreType.DMA,
        pltpu.SemaphoreType.DMA,
        pltpu.SemaphoreType.DMA,
    ],
)
def _sc_body(fh_idx, tk_idx, nm_idx, nmh_idx, fb_idx, ftab, ttab, ntab,
             ieb_out, his_out, hsum_out,
             fidx0, fidx1, tidx0, tidx1, nidx_v, nhidx_v, fbidx_v,
             fg0, fg1, tg0, tg1, hisb0, hisb1, hsumb,
             n1, n2, fbg, iebb,
             semA, semB, semWA, semWB, semH, semE):
  wid = lax.axis_index("s") * NC + lax.axis_index("c")
  wbase = wid * PB
  fidx = (fidx0, fidx1)
  tidx = (tidx0, tidx1)
  fg = (fg0, fg1)
  tg = (tg0, tg1)
  hisb = (hisb0, hisb1)
  semG = (semA, semB)
  semW = (semWA, semWB)
  zero = jnp.zeros((16,), jnp.float32)

  def fire(si, buf):
    base = wbase + si * NB
    pltpu.sync_copy(fh_idx.at[pl.ds(base * T * F, FH_PER)], fidx[buf])
    pltpu.sync_copy(tk_idx.at[pl.ds(base * T, TK_PER)], tidx[buf])
    for off, sz in _chunks(FH_PER):
      pltpu.async_copy(ftab.at[fidx[buf].at[pl.ds(off, sz)]],
                       fg[buf].at[pl.ds(off, sz)], semG[buf])
    for off, sz in _chunks(TK_PER):
      pltpu.async_copy(ttab.at[tidx[buf].at[pl.ds(off, sz)]],
                       tg[buf].at[pl.ds(off, sz)], semG[buf])

  def drain_gathers(buf):
    pltpu.make_async_copy(ftab.at[pl.ds(0, FH_PER)], fg[buf], semG[buf]).wait()
    pltpu.make_async_copy(ttab.at[pl.ds(0, TK_PER)], tg[buf], semG[buf]).wait()

  def compute_chunk(si, buf, hrow):
    fgb, tgb, hb = fg[buf], tg[buf], hisb[buf]

    for b in range(NB):
      def tbody(t, acc):
        a0, a1, a2, a3 = acc
        p = b * T + t
        v0 = (fgb[4 * p, pl.ds(0, 16)] + fgb[4 * p + 1, pl.ds(0, 16)] +
              fgb[4 * p + 2, pl.ds(0, 16)] + fgb[4 * p + 3, pl.ds(0, 16)])
        v1 = (fgb[4 * p, pl.ds(16, 16)] + fgb[4 * p + 1, pl.ds(16, 16)] +
              fgb[4 * p + 2, pl.ds(16, 16)] + fgb[4 * p + 3, pl.ds(16, 16)])
        w0 = tgb[p, pl.ds(0, 16)]
        w1 = tgb[p, pl.ds(16, 16)]
        hb[b, t, pl.ds(0, 16)] = v0
        hb[b, t, pl.ds(16, 16)] = v1
        hb[b, t, pl.ds(32, 16)] = w0
        hb[b, t, pl.ds(48, 16)] = w1
        return (a0 + v0, a1 + v1, a2 + w0, a3 + w1)
      a0, a1, a2, a3 = lax.fori_loop(0, T, tbody, (zero, zero, zero, zero))
      hsumb[hrow + b, pl.ds(0, 16)] = a0
      hsumb[hrow + b, pl.ds(16, 16)] = a1
      hsumb[hrow + b, pl.ds(32, 16)] = a2
      hsumb[hrow + b, pl.ds(48, 16)] = a3

    pltpu.async_copy(hb, his_out.at[pl.ds(wbase + si * NB, NB)], semW[buf])

  pltpu.sync_copy(nm_idx.at[pl.ds(wbase, PB)], nidx_v)
  pltpu.sync_copy(nmh_idx.at[pl.ds(wbase, PB)], nhidx_v)
  pltpu.sync_copy(fb_idx.at[pl.ds(wbase * F, PB * F)], fbidx_v)
  pltpu.async_copy(ntab.at[nidx_v], n1, semE)
  pltpu.async_copy(ntab.at[nhidx_v], n2, semE)
  for off, sz in _chunks(PB * F):
    pltpu.async_copy(ftab.at[fbidx_v.at[pl.ds(off, sz)]],
                     fbg.at[pl.ds(off, sz)], semE)
  fire(0, 0)
  fire(1, 1)

  pltpu.make_async_copy(ntab.at[pl.ds(0, PB)], n1, semE).wait()
  pltpu.make_async_copy(ntab.at[pl.ds(0, PB)], n2, semE).wait()
  pltpu.make_async_copy(ftab.at[pl.ds(0, PB * F)], fbg, semE).wait()
  for j in range(2):
    def ibody(r, c):
      b = j * IH + r
      for h in (0, 16):
        iebb[r, pl.ds(h, 16)] = n1[b, pl.ds(h, 16)]
        iebb[r, pl.ds(D + h, 16)] = n2[b, pl.ds(h, 16)]
        fv = (fbg[4 * b, pl.ds(h, 16)] + fbg[4 * b + 1, pl.ds(h, 16)] +
              fbg[4 * b + 2, pl.ds(h, 16)] + fbg[4 * b + 3, pl.ds(h, 16)])
        iebb[r, pl.ds(2 * D + h, 16)] = fv
      return c
    lax.fori_loop(0, IH, ibody, 0)
    pltpu.sync_copy(iebb, ieb_out.at[pl.ds(wbase + j * IH, IH)])

  def pairbody(pi, carry):
    @pl.when(pi > 0)
    def _():
      pltpu.make_async_copy(hisb[0],
                            his_out.at[pl.ds(wbase, NB)],
                            semW[0]).wait()
      pltpu.make_async_copy(hsumb,
                            hsum_out.at[pl.ds(wbase, 2 * NB)],
                            semH).wait()
    drain_gathers(0)
    compute_chunk(2 * pi, 0, 0)

    @pl.when(pi < NPAIR - 1)
    def _():
      fire(2 * pi + 2, 0)

    @pl.when(pi > 0)
    def _():
      pltpu.make_async_copy(hisb[1],
                            his_out.at[pl.ds(wbase, NB)],
                            semW[1]).wait()
    drain_gathers(1)
    compute_chunk(2 * pi + 1, 1, NB)
    pltpu.async_copy(hsumb, hsum_out.at[pl.ds(wbase + pi * 2 * NB, 2 * NB)],
                     semH)

    @pl.when(pi < NPAIR - 1)
    def _():
      fire(2 * pi + 3, 1)
    return carry

  lax.fori_loop(0, NPAIR, pairbody, 0)

  pltpu.make_async_copy(hisb[0], his_out.at[pl.ds(wbase, NB)],
                        semW[0]).wait()
  pltpu.make_async_copy(hisb[1], his_out.at[pl.ds(wbase, NB)],
                        semW[1]).wait()
  pltpu.make_async_copy(hsumb, hsum_out.at[pl.ds(wbase, 2 * NB)],
                        semH).wait()


@jax.jit
def kernel(nameid_batch, funcid_batch, nameid_his_batch, funcid_his_batch,
           trackid_his_batch, nameid_emb, funcid_emb, trackid_emb):
  ieb, his, hsum = _sc_body(
      funcid_his_batch.reshape(B * T * F),
      trackid_his_batch.reshape(B * T),
      nameid_batch,
      nameid_his_batch,
      funcid_batch.reshape(B * F),
      funcid_emb, trackid_emb, nameid_emb)
  return ieb, his, hsum
